# Initial kernel scaffold; baseline (speedup 1.0000x reference)
import jax
import jax.numpy as jnp
from jax.experimental import pallas as pl


def kernel(x, edge_index, edge_attr, Ws, att_src, att_dst, W_edges, att_edges, biases):
    raise NotImplementedError("write your pallas kernel here")


# speedup vs baseline: 16.9592x; 16.9592x over previous
import jax
import jax.numpy as jnp
from jax import lax
from jax.experimental import pallas as pl
from jax.experimental.pallas import tpu as pltpu
from jax.experimental.pallas import tpu_sc as plsc

N_NODES = 10000
NPAD = 10240
N_EDGES = 320000
D = 128
EDGE_DIM = 16
H = 5
NC = 2
NS = 16
E_PER_CORE = N_EDGES // NC
E_PER_SUB = E_PER_CORE // NS
CH = 80
NCHUNK = E_PER_SUB // CH
ROWS_PER_SUB = NPAD // NS
NB = 10
BN = NPAD // NB
EB = 160
BE = N_EDGES // EB


def _prep_body(x_ref, w_ref, apad_ref, h_ref, aux_ref):
    xw = jnp.dot(x_ref[...], w_ref[0], preferred_element_type=jnp.float32)
    h_ref[0] = xw
    aux_ref[0] = jnp.dot(xw, apad_ref[0], preferred_element_type=jnp.float32)


def _aer_body(ea_ref, m_ref, out_ref):
    col = lax.broadcasted_iota(jnp.int32, (BE, EDGE_DIM), 1)
    out_ref[...] = (jnp.dot(ea_ref[...], m_ref[...],
                            preferred_element_type=jnp.float32)
                    + jnp.where(col == H, 1.0, 0.0))


def _sc_body(src_hbm, dst_hbm, aert_hbm, aer16_hbm, asrc_hbm, adst_hbm,
             shift_hbm, h_hbm, z128_hbm, z16_hbm,
             acc_out, den_out, p1_out,
             src_v, dst_v, soff_v, aer_v, ex_v, rows_v, exrow_v, aer16_v,
             asrc_t, adst_t, shift_v, acc_sh, seg_sh):
    cid = lax.axis_index("c")
    sid = lax.axis_index("s")
    ebase = cid * E_PER_CORE + sid * E_PER_SUB
    rbase = sid * ROWS_PER_SUB

    pltpu.sync_copy(z16_hbm, seg_sh.at[pl.ds(rbase, ROWS_PER_SUB)])
    plsc.subcore_barrier()

    def p1_body(i, carry):
        off = ebase + i * CH
        pltpu.sync_copy(dst_hbm.at[pl.ds(off, CH)], dst_v)
        pltpu.sync_copy(aer16_hbm.at[pl.ds(off, CH)], aer16_v)
        pltpu.sync_copy(aer16_v, seg_sh.at[dst_v], add=True)
        return carry

    lax.fori_loop(0, NCHUNK, p1_body, 0)
    plsc.subcore_barrier()
    pltpu.sync_copy(seg_sh.at[pl.ds(rbase, ROWS_PER_SUB)],
                    p1_out.at[pl.ds(cid * NPAD + rbase, ROWS_PER_SUB)])
    plsc.subcore_barrier()

    pltpu.sync_copy(shift_hbm, shift_v)
    shifts = shift_v[...]

    for h in range(H):
        pltpu.sync_copy(z128_hbm, acc_sh.at[pl.ds(rbase, ROWS_PER_SUB)])
        pltpu.sync_copy(z16_hbm, seg_sh.at[pl.ds(rbase, ROWS_PER_SUB)])
        pltpu.sync_copy(asrc_hbm.at[pl.ds(h * NPAD, NPAD)], asrc_t)
        pltpu.sync_copy(adst_hbm.at[pl.ds(h * NPAD, NPAD)], adst_t)
        plsc.subcore_barrier()
        sh = shifts[h]

        def chunk_body(i, carry, h=h, sh=sh):
            off = ebase + i * CH
            pltpu.sync_copy(src_hbm.at[pl.ds(off, CH)], src_v)
            pltpu.sync_copy(dst_hbm.at[pl.ds(off, CH)], dst_v)
            pltpu.sync_copy(aert_hbm.at[pl.ds(h * N_EDGES + off, CH)], aer_v)

            def vec_body(j, c2):
                sl = pl.ds(j * 16, 16)
                si = src_v[sl]
                di = dst_v[sl]
                a = (plsc.load_gather(asrc_t, [si]) +
                     plsc.load_gather(adst_t, [di]) + aer_v[sl])
                a = jnp.maximum(a, 0.2 * a)
                ex_v[sl] = jnp.exp(a - sh)
                soff_v[sl] = si + h * NPAD
                return c2

            lax.fori_loop(0, CH // 16, vec_body, 0)
            pltpu.sync_copy(h_hbm.at[soff_v], rows_v)

            def row_body(r, c2):
                e = plsc.load_gather(ex_v, [jnp.full((16,), r, jnp.int32)])
                exrow_v[r, pl.ds(0, 16)] = e
                for j in range(D // 16):
                    sl = pl.ds(j * 16, 16)
                    rows_v[r, sl] = rows_v[r, sl] * e
                return c2

            lax.fori_loop(0, CH, row_body, 0)
            pltpu.sync_copy(rows_v, acc_sh.at[dst_v], add=True)
            pltpu.sync_copy(exrow_v, seg_sh.at[dst_v], add=True)
            return carry

        lax.fori_loop(0, NCHUNK, chunk_body, 0)
        plsc.subcore_barrier()
        pltpu.sync_copy(
            acc_sh.at[pl.ds(rbase, ROWS_PER_SUB)],
            acc_out.at[pl.ds((cid * H + h) * NPAD + rbase, ROWS_PER_SUB)])
        pltpu.sync_copy(
            seg_sh.at[pl.ds(rbase, ROWS_PER_SUB)],
            den_out.at[pl.ds((cid * H + h) * NPAD + rbase, ROWS_PER_SUB)])
        plsc.subcore_barrier()


def _asm_body(acc_ref, den_ref, p1_ref, h_ref, aux_ref, shift_ref, bias_ref,
              out_ref):
    p1s = p1_ref[0] + p1_ref[1]
    deg = jnp.maximum(p1s[:, H:H + 1], 1.0)
    for h in range(H):
        num = acc_ref[0, h] + acc_ref[1, h]
        den = den_ref[0, h, :, 0:1] + den_ref[1, h, :, 0:1]
        hb = h_ref[h]
        ael = p1s[:, h:h + 1] / deg
        al = aux_ref[h, :, 0:1] + aux_ref[h, :, 1:2] + ael
        al = jnp.maximum(al, 0.2 * al)
        exl = jnp.exp(al - shift_ref[0, h])
        out_ref[:, h * D:(h + 1) * D] = (
            (num + exl * hb) / (den + exl + 1e-16) + bias_ref[h:h + 1, :])


def kernel(x, edge_index, edge_attr, Ws, att_src, att_dst, W_edges,
           att_edges, biases):
    ei = edge_index.astype(jnp.int32)
    src = ei[0]
    dst = ei[1]
    xpad = jnp.zeros((NPAD, D), jnp.float32).at[:N_NODES].set(x)

    Apad = jnp.zeros((H, D, 8), jnp.float32)
    Apad = Apad.at[:, :, 0].set(att_src).at[:, :, 1].set(att_dst)
    e_vec = jnp.einsum('heo,ho->he', W_edges, att_edges)
    M16 = jnp.zeros((EDGE_DIM, EDGE_DIM), jnp.float32).at[:, :H].set(e_vec.T)

    hmat, aux = pl.pallas_call(
        _prep_body,
        grid=(H, NB),
        in_specs=[
            pl.BlockSpec((BN, D), lambda h, nb: (nb, 0)),
            pl.BlockSpec((1, D, D), lambda h, nb: (h, 0, 0)),
            pl.BlockSpec((1, D, 8), lambda h, nb: (h, 0, 0)),
        ],
        out_specs=[
            pl.BlockSpec((1, BN, D), lambda h, nb: (h, nb, 0)),
            pl.BlockSpec((1, BN, 8), lambda h, nb: (h, nb, 0)),
        ],
        out_shape=[
            jax.ShapeDtypeStruct((H, NPAD, D), jnp.float32),
            jax.ShapeDtypeStruct((H, NPAD, 8), jnp.float32),
        ],
    )(xpad, Ws, Apad)

    aer16 = pl.pallas_call(
        _aer_body,
        grid=(EB,),
        in_specs=[
            pl.BlockSpec((BE, EDGE_DIM), lambda i: (i, 0)),
            pl.BlockSpec((EDGE_DIM, EDGE_DIM), lambda i: (0, 0)),
        ],
        out_specs=pl.BlockSpec((BE, EDGE_DIM), lambda i: (i, 0)),
        out_shape=jax.ShapeDtypeStruct((N_EDGES, EDGE_DIM), jnp.float32),
    )(edge_attr, M16)

    asrc_flat = aux[:, :, 0].reshape(-1)
    adst_flat = aux[:, :, 1].reshape(-1)
    aer5 = aer16[:, :H]
    aert_flat = aer5.T.reshape(-1)
    shift5 = (jnp.max(aux[:, :, 0], axis=1) + jnp.max(aux[:, :, 1], axis=1)
              + jnp.maximum(jnp.max(aer5, axis=0), 0.0))
    shift16 = jnp.concatenate([shift5, jnp.zeros((11,), jnp.float32)])
    z128 = jnp.zeros((ROWS_PER_SUB, D), jnp.float32)
    z16 = jnp.zeros((ROWS_PER_SUB, EDGE_DIM), jnp.float32)
    h_flat = hmat.reshape(H * NPAD, D)

    sc = pl.kernel(
        _sc_body,
        mesh=plsc.VectorSubcoreMesh(core_axis_name="c", subcore_axis_name="s"),
        compiler_params=pltpu.CompilerParams(needs_layout_passes=False,
                                             use_tc_tiling_on_sc=False),
        out_type=[
            jax.ShapeDtypeStruct((NC * H * NPAD, D), jnp.float32),
            jax.ShapeDtypeStruct((NC * H * NPAD, EDGE_DIM), jnp.float32),
            jax.ShapeDtypeStruct((NC * NPAD, EDGE_DIM), jnp.float32),
        ],
        scratch_types=[
            pltpu.VMEM((CH,), jnp.int32),
            pltpu.VMEM((CH,), jnp.int32),
            pltpu.VMEM((CH,), jnp.int32),
            pltpu.VMEM((CH,), jnp.float32),
            pltpu.VMEM((CH,), jnp.float32),
            pltpu.VMEM((CH, D), jnp.float32),
            pltpu.VMEM((CH, EDGE_DIM), jnp.float32),
            pltpu.VMEM((CH, EDGE_DIM), jnp.float32),
            pltpu.VMEM((NPAD,), jnp.float32),
            pltpu.VMEM((NPAD,), jnp.float32),
            pltpu.VMEM((16,), jnp.float32),
            pltpu.VMEM_SHARED((NPAD, D), jnp.float32),
            pltpu.VMEM_SHARED((NPAD, EDGE_DIM), jnp.float32),
        ],
    )
    acc, den, p1 = sc(src, dst, aert_flat, aer16, asrc_flat, adst_flat,
                      shift16, h_flat, z128, z16)
    acc = acc.reshape(NC, H, NPAD, D)
    den = den.reshape(NC, H, NPAD, EDGE_DIM)
    p1 = p1.reshape(NC, NPAD, EDGE_DIM)

    out = pl.pallas_call(
        _asm_body,
        grid=(NB,),
        in_specs=[
            pl.BlockSpec((NC, H, BN, D), lambda nb: (0, 0, nb, 0)),
            pl.BlockSpec((NC, H, BN, EDGE_DIM), lambda nb: (0, 0, nb, 0)),
            pl.BlockSpec((NC, BN, EDGE_DIM), lambda nb: (0, nb, 0)),
            pl.BlockSpec((H, BN, D), lambda nb: (0, nb, 0)),
            pl.BlockSpec((H, BN, 8), lambda nb: (0, nb, 0)),
            pl.BlockSpec((1, 16), lambda nb: (0, 0)),
            pl.BlockSpec((H, D), lambda nb: (0, 0)),
        ],
        out_specs=pl.BlockSpec((BN, H * D), lambda nb: (nb, 0)),
        out_shape=jax.ShapeDtypeStruct((NPAD, H * D), jnp.float32),
    )(acc, den, p1, hmat, aux, shift16.reshape(1, 16), biases)
    return out[:N_NODES]


# speedup vs baseline: 17.0066x; 1.0028x over previous
import jax
import jax.numpy as jnp
from jax import lax
from jax.experimental import pallas as pl
from jax.experimental.pallas import tpu as pltpu
from jax.experimental.pallas import tpu_sc as plsc

N_NODES = 10000
NPAD = 10240
N_EDGES = 320000
D = 128
EDGE_DIM = 16
H = 5
NC = 2
NS = 16
E_PER_CORE = N_EDGES // NC
E_PER_SUB = E_PER_CORE // NS
CH = 80
NCHUNK = E_PER_SUB // CH
ROWS_PER_SUB = NPAD // NS
NB = 10
BN = NPAD // NB
EB = 160
BE = N_EDGES // EB


def _prep_body(x_ref, w_ref, apad_ref, h_ref, aux_ref):
    xw = jnp.dot(x_ref[...], w_ref[0], preferred_element_type=jnp.float32)
    h_ref[0] = xw
    aux_ref[0] = jnp.dot(xw, apad_ref[0], preferred_element_type=jnp.float32)


def _aer_body(ea_ref, m_ref, out_ref):
    col = lax.broadcasted_iota(jnp.int32, (BE, EDGE_DIM), 1)
    out_ref[...] = (jnp.dot(ea_ref[...], m_ref[...],
                            preferred_element_type=jnp.float32)
                    + jnp.where(col == H, 1.0, 0.0))


def _sc_body(eb_hbm, aer16_hbm, asrc_hbm, adst_hbm,
             shift_hbm, h_hbm, z128_hbm, z16_hbm,
             acc_out, den_out, p1_out,
             dst_v, soff_v, ex_v, rows_v, exrow_v, aer16_v, eb_v,
             asrc_t, adst_t, shift_v, acc_sh, seg_sh):
    cid = lax.axis_index("c")
    sid = lax.axis_index("s")
    ebase = cid * E_PER_CORE + sid * E_PER_SUB
    rbase = sid * ROWS_PER_SUB
    lane = lax.iota(jnp.int32, 16)

    pltpu.sync_copy(z16_hbm, seg_sh.at[pl.ds(rbase, ROWS_PER_SUB)])
    plsc.subcore_barrier()

    def p1_body(i, carry):
        off = ebase + i * CH
        pltpu.sync_copy(eb_hbm.at[pl.ds(off, CH)], eb_v)
        pltpu.sync_copy(aer16_hbm.at[pl.ds(off, CH)], aer16_v)

        def idx_body(j, c2):
            rows16 = lane + j * 16
            dst_v[pl.ds(j * 16, 16)] = plsc.load_gather(
                eb_v, [rows16, jnp.full((16,), 1, jnp.int32)])
            return c2

        lax.fori_loop(0, CH // 16, idx_body, 0)
        pltpu.sync_copy(aer16_v, seg_sh.at[dst_v], add=True)
        return carry

    lax.fori_loop(0, NCHUNK, p1_body, 0)
    plsc.subcore_barrier()
    pltpu.sync_copy(seg_sh.at[pl.ds(rbase, ROWS_PER_SUB)],
                    p1_out.at[pl.ds(cid * NPAD + rbase, ROWS_PER_SUB)])
    plsc.subcore_barrier()

    pltpu.sync_copy(shift_hbm, shift_v)
    shifts = shift_v[...]

    for h in range(H):
        pltpu.sync_copy(z128_hbm, acc_sh.at[pl.ds(rbase, ROWS_PER_SUB)])
        pltpu.sync_copy(z16_hbm, seg_sh.at[pl.ds(rbase, ROWS_PER_SUB)])
        pltpu.sync_copy(asrc_hbm.at[pl.ds(h * NPAD, NPAD)], asrc_t)
        pltpu.sync_copy(adst_hbm.at[pl.ds(h * NPAD, NPAD)], adst_t)
        plsc.subcore_barrier()
        sh = shifts[h]

        def chunk_body(i, carry, h=h, sh=sh):
            off = ebase + i * CH
            pltpu.sync_copy(eb_hbm.at[pl.ds(off, CH)], eb_v)

            def vec_body(j, c2):
                sl = pl.ds(j * 16, 16)
                rows16 = lane + j * 16
                si = plsc.load_gather(
                    eb_v, [rows16, jnp.full((16,), 0, jnp.int32)])
                di = plsc.load_gather(
                    eb_v, [rows16, jnp.full((16,), 1, jnp.int32)])
                ai = plsc.load_gather(
                    eb_v, [rows16, jnp.full((16,), 2 + h, jnp.int32)])
                a = (plsc.load_gather(asrc_t, [si]) +
                     plsc.load_gather(adst_t, [di]) +
                     plsc.bitcast(ai, jnp.float32))
                a = jnp.maximum(a, 0.2 * a)
                ex_v[sl] = jnp.exp(a - sh)
                soff_v[sl] = si + h * NPAD
                dst_v[sl] = di
                return c2

            lax.fori_loop(0, CH // 16, vec_body, 0)
            pltpu.sync_copy(h_hbm.at[soff_v], rows_v)

            def row_body(r, c2):
                e = plsc.load_gather(ex_v, [jnp.full((16,), r, jnp.int32)])
                exrow_v[r, pl.ds(0, 16)] = e
                for j in range(D // 16):
                    sl = pl.ds(j * 16, 16)
                    rows_v[r, sl] = rows_v[r, sl] * e
                return c2

            lax.fori_loop(0, CH, row_body, 0)
            pltpu.sync_copy(rows_v, acc_sh.at[dst_v], add=True)
            pltpu.sync_copy(exrow_v, seg_sh.at[dst_v], add=True)
            return carry

        lax.fori_loop(0, NCHUNK, chunk_body, 0)
        plsc.subcore_barrier()
        pltpu.sync_copy(
            acc_sh.at[pl.ds(rbase, ROWS_PER_SUB)],
            acc_out.at[pl.ds((cid * H + h) * NPAD + rbase, ROWS_PER_SUB)])
        pltpu.sync_copy(
            seg_sh.at[pl.ds(rbase, ROWS_PER_SUB)],
            den_out.at[pl.ds((cid * H + h) * NPAD + rbase, ROWS_PER_SUB)])
        plsc.subcore_barrier()


def _asm_body(acc_ref, den_ref, p1_ref, h_ref, aux_ref, shift_ref, bias_ref,
              out_ref):
    p1s = p1_ref[0] + p1_ref[1]
    deg = jnp.maximum(p1s[:, H:H + 1], 1.0)
    for h in range(H):
        num = acc_ref[0, h] + acc_ref[1, h]
        den = den_ref[0, h, :, 0:1] + den_ref[1, h, :, 0:1]
        hb = h_ref[h]
        ael = p1s[:, h:h + 1] / deg
        al = aux_ref[h, :, 0:1] + aux_ref[h, :, 1:2] + ael
        al = jnp.maximum(al, 0.2 * al)
        exl = jnp.exp(al - shift_ref[0, h])
        out_ref[:, h * D:(h + 1) * D] = (
            (num + exl * hb) / (den + exl + 1e-16) + bias_ref[h:h + 1, :])


def kernel(x, edge_index, edge_attr, Ws, att_src, att_dst, W_edges,
           att_edges, biases):
    ei = edge_index.astype(jnp.int32)
    src = ei[0]
    dst = ei[1]
    xpad = jnp.zeros((NPAD, D), jnp.float32).at[:N_NODES].set(x)

    Apad = jnp.zeros((H, D, 8), jnp.float32)
    Apad = Apad.at[:, :, 0].set(att_src).at[:, :, 1].set(att_dst)
    e_vec = jnp.einsum('heo,ho->he', W_edges, att_edges)
    M16 = jnp.zeros((EDGE_DIM, EDGE_DIM), jnp.float32).at[:, :H].set(e_vec.T)

    hmat, aux = pl.pallas_call(
        _prep_body,
        grid=(H, NB),
        in_specs=[
            pl.BlockSpec((BN, D), lambda h, nb: (nb, 0)),
            pl.BlockSpec((1, D, D), lambda h, nb: (h, 0, 0)),
            pl.BlockSpec((1, D, 8), lambda h, nb: (h, 0, 0)),
        ],
        out_specs=[
            pl.BlockSpec((1, BN, D), lambda h, nb: (h, nb, 0)),
            pl.BlockSpec((1, BN, 8), lambda h, nb: (h, nb, 0)),
        ],
        out_shape=[
            jax.ShapeDtypeStruct((H, NPAD, D), jnp.float32),
            jax.ShapeDtypeStruct((H, NPAD, 8), jnp.float32),
        ],
    )(xpad, Ws, Apad)

    aer16 = pl.pallas_call(
        _aer_body,
        grid=(EB,),
        in_specs=[
            pl.BlockSpec((BE, EDGE_DIM), lambda i: (i, 0)),
            pl.BlockSpec((EDGE_DIM, EDGE_DIM), lambda i: (0, 0)),
        ],
        out_specs=pl.BlockSpec((BE, EDGE_DIM), lambda i: (i, 0)),
        out_shape=jax.ShapeDtypeStruct((N_EDGES, EDGE_DIM), jnp.float32),
    )(edge_attr, M16)

    asrc_flat = aux[:, :, 0].reshape(-1)
    adst_flat = aux[:, :, 1].reshape(-1)
    aer5 = aer16[:, :H]
    shift5 = (jnp.max(aux[:, :, 0], axis=1) + jnp.max(aux[:, :, 1], axis=1)
              + jnp.maximum(jnp.max(aer5, axis=0), 0.0))
    shift16 = jnp.concatenate([shift5, jnp.zeros((11,), jnp.float32)])
    z128 = jnp.zeros((ROWS_PER_SUB, D), jnp.float32)
    z16 = jnp.zeros((ROWS_PER_SUB, EDGE_DIM), jnp.float32)
    h_flat = hmat.reshape(H * NPAD, D)

    sc = pl.kernel(
        _sc_body,
        mesh=plsc.VectorSubcoreMesh(core_axis_name="c", subcore_axis_name="s"),
        compiler_params=pltpu.CompilerParams(needs_layout_passes=False,
                                             use_tc_tiling_on_sc=False),
        out_type=[
            jax.ShapeDtypeStruct((NC * H * NPAD, D), jnp.float32),
            jax.ShapeDtypeStruct((NC * H * NPAD, EDGE_DIM), jnp.float32),
            jax.ShapeDtypeStruct((NC * NPAD, EDGE_DIM), jnp.float32),
        ],
        scratch_types=[
            pltpu.VMEM((CH,), jnp.int32),
            pltpu.VMEM((CH,), jnp.int32),
            pltpu.VMEM((CH,), jnp.float32),
            pltpu.VMEM((CH, D), jnp.float32),
            pltpu.VMEM((CH, EDGE_DIM), jnp.float32),
            pltpu.VMEM((CH, EDGE_DIM), jnp.float32),
            pltpu.VMEM((CH, 8), jnp.int32),
            pltpu.VMEM((NPAD,), jnp.float32),
            pltpu.VMEM((NPAD,), jnp.float32),
            pltpu.VMEM((16,), jnp.float32),
            pltpu.VMEM_SHARED((NPAD, D), jnp.float32),
            pltpu.VMEM_SHARED((NPAD, EDGE_DIM), jnp.float32),
        ],
    )
    ebuf = jnp.concatenate(
        [src[:, None], dst[:, None],
         lax.bitcast_convert_type(aer5, jnp.int32),
         jnp.zeros((N_EDGES, 1), jnp.int32)], axis=1)
    acc, den, p1 = sc(ebuf, aer16, asrc_flat, adst_flat,
                      shift16, h_flat, z128, z16)
    acc = acc.reshape(NC, H, NPAD, D)
    den = den.reshape(NC, H, NPAD, EDGE_DIM)
    p1 = p1.reshape(NC, NPAD, EDGE_DIM)

    out = pl.pallas_call(
        _asm_body,
        grid=(NB,),
        in_specs=[
            pl.BlockSpec((NC, H, BN, D), lambda nb: (0, 0, nb, 0)),
            pl.BlockSpec((NC, H, BN, EDGE_DIM), lambda nb: (0, 0, nb, 0)),
            pl.BlockSpec((NC, BN, EDGE_DIM), lambda nb: (0, nb, 0)),
            pl.BlockSpec((H, BN, D), lambda nb: (0, nb, 0)),
            pl.BlockSpec((H, BN, 8), lambda nb: (0, nb, 0)),
            pl.BlockSpec((1, 16), lambda nb: (0, 0)),
            pl.BlockSpec((H, D), lambda nb: (0, 0)),
        ],
        out_specs=pl.BlockSpec((BN, H * D), lambda nb: (nb, 0)),
        out_shape=jax.ShapeDtypeStruct((NPAD, H * D), jnp.float32),
    )(acc, den, p1, hmat, aux, shift16.reshape(1, 16), biases)
    return out[:N_NODES]
